# baseline (speedup 1.0000x reference)
import functools
import jax
import jax.numpy as jnp
import numpy as np
from jax import lax
from jax.experimental import pallas as pl
from jax.experimental.pallas import tpu as pltpu
from jax.experimental.pallas import tpu_sc as plsc

T, N, F = 3, 10000, 128
E, EL = 320000, 100000
H = 8
D = 64
HID = 64
TN = T * N

NC, NS = 2, 16
NW = NC * NS
CH = 128
SUP = 5

TE = T * E
NCHE = TE // CH
SUP_L = 4
TEL = T * EL
NCHL_PAD = 2432
TELP = NCHL_PAD * CH
DEC_W = NCHL_PAD // NW

RPT = TN // NS

_SC_PARAMS = pltpu.CompilerParams(
    needs_layout_passes=False, use_tc_tiling_on_sc=False)


def _mesh():
    return plsc.VectorSubcoreMesh(core_axis_name="c", subcore_axis_name="s")


def _edge_pass1(aa, gsrc2d, gdst2d, zden):
    @functools.partial(
        pl.kernel,
        mesh=_mesh(),
        compiler_params=_SC_PARAMS,
        out_type=(jax.ShapeDtypeStruct((NCHE, CH, H), jnp.float32),
                  jax.ShapeDtypeStruct((NC, TN, H), jnp.float32)),
        scratch_types=[
            pltpu.VMEM((SUP, CH), jnp.int32),
            pltpu.VMEM((SUP, CH), jnp.int32),
            pltpu.VMEM((SUP, CH, 2 * H), jnp.float32),
            pltpu.VMEM((SUP, CH, 2 * H), jnp.float32),
            pltpu.VMEM((SUP, CH, H), jnp.float32),
            pltpu.VMEM_SHARED((TN, H), jnp.float32),
            pltpu.SemaphoreType.DMA,
        ],
    )
    def k(aa_hbm, gsrc_hbm, gdst_hbm, zden_hbm, ex_hbm, denp_hbm,
          isrc_v, idst_v, srow_v, drow_v, ex_v, den_sh, sem):
        cid = lax.axis_index("c")
        sid = lax.axis_index("s")
        wid = cid * NS + sid
        pltpu.sync_copy(zden_hbm.at[pl.ds(sid * RPT, RPT)],
                        den_sh.at[pl.ds(sid * RPT, RPT)])
        plsc.subcore_barrier()
        iota = lax.iota(jnp.int32, 16)
        nsup = 46 + jnp.where(wid < 28, 1, 0)
        base_chunk = wid * 230 + jnp.minimum(wid, 28) * SUP

        def step(s, _):
            ch0 = base_chunk + s * SUP
            pltpu.sync_copy(gsrc_hbm.at[pl.ds(ch0, SUP)], isrc_v)
            pltpu.sync_copy(gdst_hbm.at[pl.ds(ch0, SUP)], idst_v)
            descs = []
            for j in range(SUP):
                descs.append(pltpu.async_copy(
                    aa_hbm.at[isrc_v.at[j]], srow_v.at[j], sem))
                descs.append(pltpu.async_copy(
                    aa_hbm.at[idst_v.at[j]], drow_v.at[j], sem))
            for dsc in descs:
                dsc.wait()
            for j in range(SUP):
                jv = jnp.full((16,), j, jnp.int32)

                def body(i, _):
                    kk = i * 16 + iota
                    cv = kk >> 3
                    hv = kk & 7
                    sv = plsc.load_gather(srow_v, [jv, cv, hv])
                    dv = plsc.load_gather(drow_v, [jv, cv, hv + 8])
                    sm = sv + dv
                    val = jnp.exp(jnp.maximum(sm, 0.2 * sm))
                    plsc.store_scatter(ex_v, [jv, cv, hv], val)
                    return 0
                lax.fori_loop(0, CH * H // 16, body, 0, unroll=4)
            pltpu.sync_copy(ex_v, ex_hbm.at[pl.ds(ch0, SUP)])
            for j in range(SUP):
                pltpu.sync_copy(ex_v.at[j], den_sh.at[idst_v.at[j]], add=True)
            return 0
        lax.fori_loop(0, nsup, step, 0)
        plsc.subcore_barrier()
        pltpu.sync_copy(den_sh.at[pl.ds(sid * RPT, RPT)],
                        denp_hbm.at[cid, pl.ds(sid * RPT, RPT)])

    return k(aa, gsrc2d, gdst2d, zden)


NCHT = E // CH
RPN = N // NS


def _edge_pass2(h_table, recip, exbuf, gsrc2d, gdst2d, ldst2d, zout):
    @functools.partial(
        pl.kernel,
        mesh=_mesh(),
        compiler_params=_SC_PARAMS,
        out_type=(jax.ShapeDtypeStruct((NCHE, CH, H), jnp.float32),
                  jax.ShapeDtypeStruct((NC, T, N, D), jnp.float32)),
        scratch_types=[
            pltpu.VMEM((SUP, CH), jnp.int32),
            pltpu.VMEM((SUP, CH), jnp.int32),
            pltpu.VMEM((SUP, CH), jnp.int32),
            pltpu.VMEM((SUP, CH, D), jnp.bfloat16),
            pltpu.VMEM((SUP, CH, D), jnp.float32),
            pltpu.VMEM((SUP, CH, H), jnp.float32),
            pltpu.VMEM((SUP, CH, H), jnp.float32),
            pltpu.VMEM_SHARED((N, D), jnp.float32),
            pltpu.SemaphoreType.DMA,
        ],
    )
    def k(h_hbm, recip_hbm, ex_hbm, gsrc_hbm, gdst_hbm, ldst_hbm, zout_hbm,
          coe_hbm, outp_hbm,
          isrc_v, idst_v, ildst_v, hrow_v, arow_v, rrow_v, ex_v, out_sh, sem):
        cid = lax.axis_index("c")
        sid = lax.axis_index("s")
        wid = cid * NS + sid
        iota = lax.iota(jnp.int32, 16)
        nsup = 15 + jnp.where(wid < 20, 1, 0)
        base_local = wid * 75 + jnp.minimum(wid, 20) * SUP

        def tbody(t, _):
            pltpu.sync_copy(zout_hbm.at[pl.ds(sid * RPN, RPN)],
                            out_sh.at[pl.ds(sid * RPN, RPN)])
            plsc.subcore_barrier()

            def step(s, _):
                ch0 = t * NCHT + base_local + s * SUP
                pltpu.sync_copy(gsrc_hbm.at[pl.ds(ch0, SUP)], isrc_v)
                pltpu.sync_copy(gdst_hbm.at[pl.ds(ch0, SUP)], idst_v)
                pltpu.sync_copy(ldst_hbm.at[pl.ds(ch0, SUP)], ildst_v)
                pltpu.sync_copy(ex_hbm.at[pl.ds(ch0, SUP)], ex_v)
                descs = []
                for j in range(SUP):
                    descs.append(pltpu.async_copy(
                        h_hbm.at[isrc_v.at[j]], hrow_v.at[j], sem))
                    descs.append(pltpu.async_copy(
                        recip_hbm.at[idst_v.at[j]], rrow_v.at[j], sem))
                for dsc in descs:
                    dsc.wait()
                for j in range(SUP):
                    jv = jnp.full((16,), j, jnp.int32)

                    def body(i, _):
                        kk = i * 16 + iota
                        cv = kk >> 3
                        hv = kk & 7
                        ev = plsc.load_gather(ex_v, [jv, cv, hv])
                        rv = plsc.load_gather(rrow_v, [jv, cv, hv])
                        plsc.store_scatter(ex_v, [jv, cv, hv], ev * rv)
                        return 0
                    lax.fori_loop(0, CH * H // 16, body, 0, unroll=4)
                pltpu.sync_copy(ex_v, coe_hbm.at[pl.ds(ch0, SUP)])
                for j in range(SUP):
                    jv = jnp.full((16,), j, jnp.int32)

                    def body2(c, _):
                        cv = jnp.full((16,), c, jnp.int32)
                        for kq in range(2):
                            hb = hrow_v[j, c, pl.ds(kq * 32, 32)]
                            lo, hi = plsc.unpack(
                                hb, format=plsc.PackFormat.INTERLEAVED)
                            av = plsc.load_gather(
                                ex_v, [jv, cv, 4 * kq + (iota >> 2)])
                            de = kq * 32 + 2 * iota
                            plsc.store_scatter(arow_v, [jv, cv, de], lo * av)
                            plsc.store_scatter(arow_v, [jv, cv, de + 1],
                                               hi * av)
                        return 0
                    lax.fori_loop(0, CH, body2, 0, unroll=2)
                    pltpu.sync_copy(arow_v.at[j], out_sh.at[ildst_v.at[j]],
                                    add=True)
                return 0
            lax.fori_loop(0, nsup, step, 0)
            plsc.subcore_barrier()
            pltpu.sync_copy(out_sh.at[pl.ds(sid * RPN, RPN)],
                            outp_hbm.at[cid, t, pl.ds(sid * RPN, RPN)])
            return 0
        lax.fori_loop(0, T, tbody, 0)

    return k(h_table, recip, exbuf, gsrc2d, gdst2d, ldst2d, zout)


def _decode(z2d, ls2d, ld2d):
    @functools.partial(
        pl.kernel,
        mesh=_mesh(),
        compiler_params=_SC_PARAMS,
        out_type=jax.ShapeDtypeStruct((NCHL_PAD, CH), jnp.float32),
        scratch_types=[
            pltpu.VMEM((SUP_L, CH), jnp.int32),
            pltpu.VMEM((SUP_L, CH), jnp.int32),
            pltpu.VMEM((SUP_L, CH, D), jnp.float32),
            pltpu.VMEM((SUP_L, CH, D), jnp.float32),
            pltpu.VMEM((SUP_L, CH, 16), jnp.float32),
            pltpu.VMEM((SUP_L, CH), jnp.float32),
            pltpu.SemaphoreType.DMA,
        ],
    )
    def k(z_hbm, ls_hbm, ld_hbm, st_hbm, il_v, id_v, zs_v, zd_v, ps_v, st_v,
          sem):
        cid = lax.axis_index("c")
        sid = lax.axis_index("s")
        wid = cid * NS + sid
        base_chunk = wid * DEC_W
        iota = lax.iota(jnp.int32, 16)

        def step(s, _):
            ch0 = base_chunk + s * SUP_L
            pltpu.sync_copy(ls_hbm.at[pl.ds(ch0, SUP_L)], il_v)
            pltpu.sync_copy(ld_hbm.at[pl.ds(ch0, SUP_L)], id_v)
            descs = []
            for j in range(SUP_L):
                descs.append(pltpu.async_copy(
                    z_hbm.at[il_v.at[j]], zs_v.at[j], sem))
                descs.append(pltpu.async_copy(
                    z_hbm.at[id_v.at[j]], zd_v.at[j], sem))
            for dsc in descs:
                dsc.wait()
            for j in range(SUP_L):
                jv = jnp.full((16,), j, jnp.int32)

                def body(c, _):
                    acc = (zs_v[j, c, pl.ds(0, 16)] *
                           zd_v[j, c, pl.ds(0, 16)])
                    for dq in range(1, 4):
                        acc += (zs_v[j, c, pl.ds(dq * 16, 16)] *
                                zd_v[j, c, pl.ds(dq * 16, 16)])
                    ps_v[j, c, :] = acc
                    return 0
                lax.fori_loop(0, CH, body, 0, unroll=2)

                def red(g, _):
                    cv = g * 16 + iota
                    tot = plsc.load_gather(
                        ps_v, [jv, cv, jnp.zeros((16,), jnp.int32)])
                    for l in range(1, 16):
                        tot += plsc.load_gather(
                            ps_v, [jv, cv, jnp.full((16,), l, jnp.int32)])
                    st_v[j, pl.ds(g * 16, 16)] = tot
                    return 0
                lax.fori_loop(0, CH // 16, red, 0, unroll=2)
            pltpu.sync_copy(st_v, st_hbm.at[pl.ds(ch0, SUP_L)])
            return 0
        lax.fori_loop(0, DEC_W // SUP_L, step, 0)

    return k(z2d, ls2d, ld2d)


BN = 1000


def _tc_project(xf, W, A):
    Fin = xf.shape[1]

    def body(x_ref, w_ref, a_ref, h_ref, aa_ref):
        h = jnp.dot(x_ref[...], w_ref[...], preferred_element_type=jnp.float32)
        h_ref[...] = h.astype(jnp.bfloat16)
        aa_ref[...] = jnp.dot(h, a_ref[...], preferred_element_type=jnp.float32)

    return pl.pallas_call(
        body,
        grid=(TN // BN,),
        in_specs=[
            pl.BlockSpec((BN, Fin), lambda i: (i, 0)),
            pl.BlockSpec((Fin, D), lambda i: (0, 0)),
            pl.BlockSpec((D, 2 * H), lambda i: (0, 0)),
        ],
        out_specs=[
            pl.BlockSpec((BN, D), lambda i: (i, 0)),
            pl.BlockSpec((BN, 2 * H), lambda i: (i, 0)),
        ],
        out_shape=[
            jax.ShapeDtypeStruct((TN, D), jnp.bfloat16),
            jax.ShapeDtypeStruct((TN, 2 * H), jnp.float32),
        ],
    )(xf, W, A)


def _tc_combine_project(outp, W, A):
    def body(p_ref, w_ref, a_ref, h_ref, aa_ref):
        s = p_ref[0] + p_ref[1]
        o = jnp.where(s > 0, s, jnp.exp(jnp.minimum(s, 0.0)) - 1.0)
        h = jnp.dot(o, w_ref[...], preferred_element_type=jnp.float32)
        h_ref[...] = h.astype(jnp.bfloat16)
        aa_ref[...] = jnp.dot(h, a_ref[...], preferred_element_type=jnp.float32)

    return pl.pallas_call(
        body,
        grid=(TN // BN,),
        in_specs=[
            pl.BlockSpec((NC, BN, D), lambda i: (0, i, 0)),
            pl.BlockSpec((D, D), lambda i: (0, 0)),
            pl.BlockSpec((D, 2 * H), lambda i: (0, 0)),
        ],
        out_specs=[
            pl.BlockSpec((BN, D), lambda i: (i, 0)),
            pl.BlockSpec((BN, 2 * H), lambda i: (i, 0)),
        ],
        out_shape=[
            jax.ShapeDtypeStruct((TN, D), jnp.bfloat16),
            jax.ShapeDtypeStruct((TN, 2 * H), jnp.float32),
        ],
    )(outp, W, A)


def _tc_recip_den(denp):
    def body(d_ref, r_ref):
        r_ref[...] = 1.0 / (d_ref[0] + d_ref[1] + 1e-16)

    BR = 3000
    return pl.pallas_call(
        body,
        grid=(TN // BR,),
        in_specs=[pl.BlockSpec((NC, BR, H), lambda i: (0, i, 0))],
        out_specs=pl.BlockSpec((BR, H), lambda i: (i, 0)),
        out_shape=jax.ShapeDtypeStruct((TN, H), jnp.float32),
    )(denp)


def _tc_lstm_attn(outp, Wi, Wh, b2, pos, Wq, Wk, Wv, HS, EX8):
    inv_sqrt_dh = 1.0 / np.sqrt(H)

    def body(p_ref, wi_ref, wh_ref, b_ref, pos_ref, wq_ref, wk_ref, wv_ref,
             hs_ref, ex_ref, z_ref):
        wi = wi_ref[...]
        wh = wh_ref[...]
        b = b_ref[...]
        hst = jnp.zeros((BN, HID), jnp.float32)
        cst = jnp.zeros((BN, HID), jnp.float32)
        seq = []
        for t in range(T):
            o2 = jnp.maximum(p_ref[0, t] + p_ref[1, t], 0.0)
            g = (jnp.dot(o2, wi, preferred_element_type=jnp.float32) +
                 jnp.dot(hst, wh, preferred_element_type=jnp.float32) + b)
            ig = jax.nn.sigmoid(g[:, 0 * HID:1 * HID])
            fg = jax.nn.sigmoid(g[:, 1 * HID:2 * HID])
            gg = jnp.tanh(g[:, 2 * HID:3 * HID])
            og = jax.nn.sigmoid(g[:, 3 * HID:4 * HID])
            cst = fg * cst + ig * gg
            hst = og * jnp.tanh(cst)
            seq.append(hst)
        hs = hs_ref[...]
        ex8 = ex_ref[...]
        xx = [seq[t] + pos_ref[t, :][None, :] for t in range(T)]
        q = [jnp.dot(xx[t], wq_ref[...], preferred_element_type=jnp.float32)
             for t in range(T)]
        kk = [jnp.dot(xx[t], wk_ref[...], preferred_element_type=jnp.float32)
              for t in range(T)]
        v = [jnp.dot(xx[t], wv_ref[...], preferred_element_type=jnp.float32)
             for t in range(T)]

        def score(qt, kt):
            return jnp.dot(q[qt] * kk[kt], hs,
                           preferred_element_type=jnp.float32) * inv_sqrt_dh

        def expand(a):
            return jnp.dot(a, ex8, preferred_element_type=jnp.float32)

        z_ref[0] = v[0]
        e0 = jnp.exp(score(1, 0))
        e1 = jnp.exp(score(1, 1))
        r = 1.0 / (e0 + e1)
        z_ref[1] = expand(e0 * r) * v[0] + expand(e1 * r) * v[1]
        f0 = jnp.exp(score(2, 0))
        f1 = jnp.exp(score(2, 1))
        f2 = jnp.exp(score(2, 2))
        rr = 1.0 / (f0 + f1 + f2)
        z_ref[2] = (expand(f0 * rr) * v[0] + expand(f1 * rr) * v[1] +
                    expand(f2 * rr) * v[2])

    return pl.pallas_call(
        body,
        grid=(N // BN,),
        in_specs=[
            pl.BlockSpec((NC, T, BN, D), lambda i: (0, 0, i, 0)),
            pl.BlockSpec((D, 4 * HID), lambda i: (0, 0)),
            pl.BlockSpec((HID, 4 * HID), lambda i: (0, 0)),
            pl.BlockSpec((1, 4 * HID), lambda i: (0, 0)),
            pl.BlockSpec((T, HID), lambda i: (0, 0)),
            pl.BlockSpec((HID, HID), lambda i: (0, 0)),
            pl.BlockSpec((HID, HID), lambda i: (0, 0)),
            pl.BlockSpec((HID, HID), lambda i: (0, 0)),
            pl.BlockSpec((HID, H), lambda i: (0, 0)),
            pl.BlockSpec((H, HID), lambda i: (0, 0)),
        ],
        out_specs=pl.BlockSpec((T, BN, HID), lambda i: (0, i, 0)),
        out_shape=jax.ShapeDtypeStruct((T, N, HID), jnp.float32),
    )(outp, Wi, Wh, b2, pos, Wq, Wk, Wv, HS, EX8)


def _blockdiag(a):
    dh = D // H
    idx = jnp.arange(D) // dh
    return jnp.zeros((D, H), jnp.float32).at[jnp.arange(D), idx].set(
        a.reshape(-1))


def kernel(x, edge_index, edge_label_index, W1, a_src1, a_dst1, W2, a_src2,
           a_dst2, Wi, Wh, b_lstm, pos_emb, Wq, Wk, Wv):
    xf = x.reshape(TN, F)
    offs = (jnp.arange(T, dtype=jnp.int32) * N)[:, None]
    gsrc2d = (edge_index[:, 0, :] + offs).reshape(NCHE, CH)
    gdst2d = (edge_index[:, 1, :] + offs).reshape(NCHE, CH)
    lsf = (edge_label_index[:, 0, :] + offs).reshape(TEL)
    ldf = (edge_label_index[:, 1, :] + offs).reshape(TEL)
    ls2d = jnp.pad(lsf, (0, TELP - TEL)).reshape(NCHL_PAD, CH)
    ld2d = jnp.pad(ldf, (0, TELP - TEL)).reshape(NCHL_PAD, CH)

    A1 = jnp.concatenate([_blockdiag(a_src1), _blockdiag(a_dst1)], axis=1)
    A2 = jnp.concatenate([_blockdiag(a_src2), _blockdiag(a_dst2)], axis=1)
    dh = D // H
    colh = jnp.arange(D) // dh
    HS = jnp.zeros((D, H), jnp.float32).at[jnp.arange(D), colh].set(1.0)
    EX8 = HS.T
    zden = jnp.zeros((TN, H), jnp.float32)
    zout = jnp.zeros((N, D), jnp.float32)
    b2 = b_lstm.reshape(1, 4 * HID)
    ldst2d = edge_index[:, 1, :].reshape(NCHE, CH)

    h1, aa1 = _tc_project(xf, W1, A1)
    ex1, den1p = _edge_pass1(aa1, gsrc2d, gdst2d, zden)
    recip1 = _tc_recip_den(den1p)
    coe1p, out1p = _edge_pass2(h1, recip1, ex1, gsrc2d, gdst2d, ldst2d, zout)
    out1p = out1p.reshape(NC, TN, D)

    h2, aa2 = _tc_combine_project(out1p, W2, A2)
    ex2, den2p = _edge_pass1(aa2, gsrc2d, gdst2d, zden)
    recip2 = _tc_recip_den(den2p)
    coe2p, out2p = _edge_pass2(h2, recip2, ex2, gsrc2d, gdst2d, ldst2d, zout)
    out2p = out2p.reshape(NC, TN, D)

    z = _tc_lstm_attn(out2p.reshape(NC, T, N, D), Wi, Wh, b2, pos_emb,
                      Wq, Wk, Wv, HS, EX8)

    st_flat = _decode(z.reshape(TN, HID), ls2d, ld2d).reshape(TELP)
    st = st_flat[:TEL].reshape(T, EL)

    coe1 = coe1p.reshape(TE, H).reshape(T, E, H)
    coe2 = coe2p.reshape(TE, H).reshape(T, E, H)
    return (st, coe1, coe2)


# speedup vs baseline: 1.1064x; 1.1064x over previous
import functools
import jax
import jax.numpy as jnp
import numpy as np
from jax import lax
from jax.experimental import pallas as pl
from jax.experimental.pallas import tpu as pltpu
from jax.experimental.pallas import tpu_sc as plsc

T, N, F = 3, 10000, 128
E, EL = 320000, 100000
H = 8
D = 64
HID = 64
TN = T * N

NC, NS = 2, 16
NW = NC * NS
CH = 128
SUP = 5

TE = T * E
NCHE = TE // CH
SUP_L = 2
TEL = T * EL
NCHL_PAD = 2368
TELP = NCHL_PAD * CH
DEC_W = NCHL_PAD // NW

RPT = TN // NS

_SC_PARAMS = pltpu.CompilerParams(
    needs_layout_passes=False, use_tc_tiling_on_sc=False)


def _mesh():
    return plsc.VectorSubcoreMesh(core_axis_name="c", subcore_axis_name="s")


def _edge_pass1(aa, gsrc2d, gdst2d, zden):
    @functools.partial(
        pl.kernel,
        mesh=_mesh(),
        compiler_params=_SC_PARAMS,
        out_type=(jax.ShapeDtypeStruct((NCHE, CH, H), jnp.float32),
                  jax.ShapeDtypeStruct((NC, TN, H), jnp.float32)),
        scratch_types=[
            pltpu.VMEM((SUP, CH), jnp.int32),
            pltpu.VMEM((SUP, CH), jnp.int32),
            pltpu.VMEM((SUP, CH, 2 * H), jnp.float32),
            pltpu.VMEM((SUP, CH, 2 * H), jnp.float32),
            pltpu.VMEM((SUP, CH, H), jnp.float32),
            pltpu.VMEM_SHARED((TN, H), jnp.float32),
            pltpu.SemaphoreType.DMA,
        ],
    )
    def k(aa_hbm, gsrc_hbm, gdst_hbm, zden_hbm, ex_hbm, denp_hbm,
          isrc_v, idst_v, srow_v, drow_v, ex_v, den_sh, sem):
        cid = lax.axis_index("c")
        sid = lax.axis_index("s")
        wid = cid * NS + sid
        pltpu.sync_copy(zden_hbm.at[pl.ds(sid * RPT, RPT)],
                        den_sh.at[pl.ds(sid * RPT, RPT)])
        plsc.subcore_barrier()
        iota = lax.iota(jnp.int32, 16)
        nsup = 46 + jnp.where(wid < 28, 1, 0)
        base_chunk = wid * 230 + jnp.minimum(wid, 28) * SUP

        def step(s, _):
            ch0 = base_chunk + s * SUP
            pltpu.sync_copy(gsrc_hbm.at[pl.ds(ch0, SUP)], isrc_v)
            pltpu.sync_copy(gdst_hbm.at[pl.ds(ch0, SUP)], idst_v)
            descs = []
            for j in range(SUP):
                descs.append(pltpu.async_copy(
                    aa_hbm.at[isrc_v.at[j]], srow_v.at[j], sem))
                descs.append(pltpu.async_copy(
                    aa_hbm.at[idst_v.at[j]], drow_v.at[j], sem))
            for dsc in descs:
                dsc.wait()
            for j in range(SUP):
                jv = jnp.full((16,), j, jnp.int32)

                def body(i, _):
                    kk = i * 16 + iota
                    cv = kk >> 3
                    hv = kk & 7
                    sv = plsc.load_gather(srow_v, [jv, cv, hv])
                    dv = plsc.load_gather(drow_v, [jv, cv, hv + 8])
                    sm = sv + dv
                    val = jnp.exp(jnp.maximum(sm, 0.2 * sm))
                    plsc.store_scatter(ex_v, [jv, cv, hv], val)
                    return 0
                lax.fori_loop(0, CH * H // 16, body, 0, unroll=4)
            pltpu.sync_copy(ex_v, ex_hbm.at[pl.ds(ch0, SUP)])
            for j in range(SUP):
                pltpu.sync_copy(ex_v.at[j], den_sh.at[idst_v.at[j]], add=True)
            return 0
        lax.fori_loop(0, nsup, step, 0)
        plsc.subcore_barrier()
        pltpu.sync_copy(den_sh.at[pl.ds(sid * RPT, RPT)],
                        denp_hbm.at[cid, pl.ds(sid * RPT, RPT)])

    return k(aa, gsrc2d, gdst2d, zden)


NCHT = E // CH
RPN = N // NS


def _edge_pass2(h_table, recip, exbuf, gsrc2d, gdst2d, ldst2d, zout):
    @functools.partial(
        pl.kernel,
        mesh=_mesh(),
        compiler_params=_SC_PARAMS,
        out_type=(jax.ShapeDtypeStruct((NCHE, CH, H), jnp.float32),
                  jax.ShapeDtypeStruct((NC, T, N, D), jnp.float32)),
        scratch_types=[
            pltpu.VMEM((SUP, CH), jnp.int32),
            pltpu.VMEM((SUP, CH), jnp.int32),
            pltpu.VMEM((SUP, CH), jnp.int32),
            pltpu.VMEM((SUP, CH, D), jnp.bfloat16),
            pltpu.VMEM((SUP, CH, D), jnp.float32),
            pltpu.VMEM((SUP, CH, H), jnp.float32),
            pltpu.VMEM((SUP, CH, H), jnp.float32),
            pltpu.VMEM_SHARED((N, D), jnp.float32),
            pltpu.SemaphoreType.DMA,
        ],
    )
    def k(h_hbm, recip_hbm, ex_hbm, gsrc_hbm, gdst_hbm, ldst_hbm, zout_hbm,
          coe_hbm, outp_hbm,
          isrc_v, idst_v, ildst_v, hrow_v, arow_v, rrow_v, ex_v, out_sh, sem):
        cid = lax.axis_index("c")
        sid = lax.axis_index("s")
        wid = cid * NS + sid
        iota = lax.iota(jnp.int32, 16)
        nsup = 15 + jnp.where(wid < 20, 1, 0)
        base_local = wid * 75 + jnp.minimum(wid, 20) * SUP

        def tbody(t, _):
            pltpu.sync_copy(zout_hbm.at[pl.ds(sid * RPN, RPN)],
                            out_sh.at[pl.ds(sid * RPN, RPN)])
            plsc.subcore_barrier()

            def step(s, _):
                ch0 = t * NCHT + base_local + s * SUP
                pltpu.sync_copy(gsrc_hbm.at[pl.ds(ch0, SUP)], isrc_v)
                pltpu.sync_copy(gdst_hbm.at[pl.ds(ch0, SUP)], idst_v)
                pltpu.sync_copy(ldst_hbm.at[pl.ds(ch0, SUP)], ildst_v)
                pltpu.sync_copy(ex_hbm.at[pl.ds(ch0, SUP)], ex_v)
                descs = []
                for j in range(SUP):
                    descs.append(pltpu.async_copy(
                        h_hbm.at[isrc_v.at[j]], hrow_v.at[j], sem))
                    descs.append(pltpu.async_copy(
                        recip_hbm.at[idst_v.at[j]], rrow_v.at[j], sem))
                for dsc in descs:
                    dsc.wait()
                for j in range(SUP):
                    jv = jnp.full((16,), j, jnp.int32)

                    def body(i, _):
                        kk = i * 16 + iota
                        cv = kk >> 3
                        hv = kk & 7
                        ev = plsc.load_gather(ex_v, [jv, cv, hv])
                        rv = plsc.load_gather(rrow_v, [jv, cv, hv])
                        plsc.store_scatter(ex_v, [jv, cv, hv], ev * rv)
                        return 0
                    lax.fori_loop(0, CH * H // 16, body, 0, unroll=4)
                pltpu.sync_copy(ex_v, coe_hbm.at[pl.ds(ch0, SUP)])
                for j in range(SUP):
                    jv = jnp.full((16,), j, jnp.int32)

                    def body2(c, _):
                        cv = jnp.full((16,), c, jnp.int32)
                        for kq in range(2):
                            hb = hrow_v[j, c, pl.ds(kq * 32, 32)]
                            lo, hi = plsc.unpack(
                                hb, format=plsc.PackFormat.INTERLEAVED)
                            av = plsc.load_gather(
                                ex_v, [jv, cv, 4 * kq + (iota >> 2)])
                            de = kq * 32 + 2 * iota
                            plsc.store_scatter(arow_v, [jv, cv, de], lo * av)
                            plsc.store_scatter(arow_v, [jv, cv, de + 1],
                                               hi * av)
                        return 0
                    lax.fori_loop(0, CH, body2, 0, unroll=2)
                    pltpu.sync_copy(arow_v.at[j], out_sh.at[ildst_v.at[j]],
                                    add=True)
                return 0
            lax.fori_loop(0, nsup, step, 0)
            plsc.subcore_barrier()
            pltpu.sync_copy(out_sh.at[pl.ds(sid * RPN, RPN)],
                            outp_hbm.at[cid, t, pl.ds(sid * RPN, RPN)])
            return 0
        lax.fori_loop(0, T, tbody, 0)

    return k(h_table, recip, exbuf, gsrc2d, gdst2d, ldst2d, zout)


def _decode(z2d, ls2d, ld2d):
    @functools.partial(
        pl.kernel,
        mesh=_mesh(),
        compiler_params=_SC_PARAMS,
        out_type=jax.ShapeDtypeStruct((NCHL_PAD, CH, 16), jnp.float32),
        scratch_types=[
            pltpu.VMEM((SUP_L, CH), jnp.int32),
            pltpu.VMEM((SUP_L, CH), jnp.int32),
            pltpu.VMEM((SUP_L, CH, D), jnp.bfloat16),
            pltpu.VMEM((SUP_L, CH, D), jnp.bfloat16),
            pltpu.VMEM((SUP_L, CH, 16), jnp.float32),
            pltpu.SemaphoreType.DMA,
        ],
    )
    def k(z_hbm, ls_hbm, ld_hbm, ps_hbm, il_v, id_v, zs_v, zd_v, ps_v, sem):
        cid = lax.axis_index("c")
        sid = lax.axis_index("s")
        wid = cid * NS + sid
        base_chunk = wid * DEC_W

        def step(s, _):
            ch0 = base_chunk + s * SUP_L
            pltpu.sync_copy(ls_hbm.at[pl.ds(ch0, SUP_L)], il_v)
            pltpu.sync_copy(ld_hbm.at[pl.ds(ch0, SUP_L)], id_v)
            descs = []
            for j in range(SUP_L):
                descs.append(pltpu.async_copy(
                    z_hbm.at[il_v.at[j]], zs_v.at[j], sem))
                descs.append(pltpu.async_copy(
                    z_hbm.at[id_v.at[j]], zd_v.at[j], sem))
            for dsc in descs:
                dsc.wait()
            for j in range(SUP_L):
                def body(c, _):
                    acc = None
                    for kq in range(2):
                        sb = zs_v[j, c, pl.ds(kq * 32, 32)]
                        db = zd_v[j, c, pl.ds(kq * 32, 32)]
                        slo, shi = plsc.unpack(
                            sb, format=plsc.PackFormat.INTERLEAVED)
                        dlo, dhi = plsc.unpack(
                            db, format=plsc.PackFormat.INTERLEAVED)
                        p = slo * dlo + shi * dhi
                        acc = p if acc is None else acc + p
                    ps_v[j, c, :] = acc
                    return 0
                lax.fori_loop(0, CH, body, 0, unroll=2)
            pltpu.sync_copy(ps_v, ps_hbm.at[pl.ds(ch0, SUP_L)])
            return 0
        lax.fori_loop(0, DEC_W // SUP_L, step, 0)

    return k(z2d, ls2d, ld2d)


BN = 1000


def _tc_project(xf, W, A):
    Fin = xf.shape[1]

    def body(x_ref, w_ref, a_ref, h_ref, aa_ref):
        h = jnp.dot(x_ref[...], w_ref[...], preferred_element_type=jnp.float32)
        h_ref[...] = h.astype(jnp.bfloat16)
        aa_ref[...] = jnp.dot(h, a_ref[...], preferred_element_type=jnp.float32)

    return pl.pallas_call(
        body,
        grid=(TN // BN,),
        in_specs=[
            pl.BlockSpec((BN, Fin), lambda i: (i, 0)),
            pl.BlockSpec((Fin, D), lambda i: (0, 0)),
            pl.BlockSpec((D, 2 * H), lambda i: (0, 0)),
        ],
        out_specs=[
            pl.BlockSpec((BN, D), lambda i: (i, 0)),
            pl.BlockSpec((BN, 2 * H), lambda i: (i, 0)),
        ],
        out_shape=[
            jax.ShapeDtypeStruct((TN, D), jnp.bfloat16),
            jax.ShapeDtypeStruct((TN, 2 * H), jnp.float32),
        ],
    )(xf, W, A)


def _tc_combine_project(outp, W, A):
    def body(p_ref, w_ref, a_ref, h_ref, aa_ref):
        s = p_ref[0] + p_ref[1]
        o = jnp.where(s > 0, s, jnp.exp(jnp.minimum(s, 0.0)) - 1.0)
        h = jnp.dot(o, w_ref[...], preferred_element_type=jnp.float32)
        h_ref[...] = h.astype(jnp.bfloat16)
        aa_ref[...] = jnp.dot(h, a_ref[...], preferred_element_type=jnp.float32)

    return pl.pallas_call(
        body,
        grid=(TN // BN,),
        in_specs=[
            pl.BlockSpec((NC, BN, D), lambda i: (0, i, 0)),
            pl.BlockSpec((D, D), lambda i: (0, 0)),
            pl.BlockSpec((D, 2 * H), lambda i: (0, 0)),
        ],
        out_specs=[
            pl.BlockSpec((BN, D), lambda i: (i, 0)),
            pl.BlockSpec((BN, 2 * H), lambda i: (i, 0)),
        ],
        out_shape=[
            jax.ShapeDtypeStruct((TN, D), jnp.bfloat16),
            jax.ShapeDtypeStruct((TN, 2 * H), jnp.float32),
        ],
    )(outp, W, A)


def _tc_recip_den(denp):
    def body(d_ref, r_ref):
        r_ref[...] = 1.0 / (d_ref[0] + d_ref[1] + 1e-16)

    BR = 3000
    return pl.pallas_call(
        body,
        grid=(TN // BR,),
        in_specs=[pl.BlockSpec((NC, BR, H), lambda i: (0, i, 0))],
        out_specs=pl.BlockSpec((BR, H), lambda i: (i, 0)),
        out_shape=jax.ShapeDtypeStruct((TN, H), jnp.float32),
    )(denp)


def _tc_lstm_attn(outp, Wi, Wh, b2, pos, Wq, Wk, Wv, HS, EX8):
    inv_sqrt_dh = 1.0 / np.sqrt(H)

    def body(p_ref, wi_ref, wh_ref, b_ref, pos_ref, wq_ref, wk_ref, wv_ref,
             hs_ref, ex_ref, z_ref):
        wi = wi_ref[...]
        wh = wh_ref[...]
        b = b_ref[...]
        hst = jnp.zeros((BN, HID), jnp.float32)
        cst = jnp.zeros((BN, HID), jnp.float32)
        seq = []
        for t in range(T):
            o2 = jnp.maximum(p_ref[0, t] + p_ref[1, t], 0.0)
            g = (jnp.dot(o2, wi, preferred_element_type=jnp.float32) +
                 jnp.dot(hst, wh, preferred_element_type=jnp.float32) + b)
            ig = jax.nn.sigmoid(g[:, 0 * HID:1 * HID])
            fg = jax.nn.sigmoid(g[:, 1 * HID:2 * HID])
            gg = jnp.tanh(g[:, 2 * HID:3 * HID])
            og = jax.nn.sigmoid(g[:, 3 * HID:4 * HID])
            cst = fg * cst + ig * gg
            hst = og * jnp.tanh(cst)
            seq.append(hst)
        hs = hs_ref[...]
        ex8 = ex_ref[...]
        xx = [seq[t] + pos_ref[t, :][None, :] for t in range(T)]
        q = [jnp.dot(xx[t], wq_ref[...], preferred_element_type=jnp.float32)
             for t in range(T)]
        kk = [jnp.dot(xx[t], wk_ref[...], preferred_element_type=jnp.float32)
              for t in range(T)]
        v = [jnp.dot(xx[t], wv_ref[...], preferred_element_type=jnp.float32)
             for t in range(T)]

        def score(qt, kt):
            return jnp.dot(q[qt] * kk[kt], hs,
                           preferred_element_type=jnp.float32) * inv_sqrt_dh

        def expand(a):
            return jnp.dot(a, ex8, preferred_element_type=jnp.float32)

        z_ref[0] = v[0].astype(jnp.bfloat16)
        e0 = jnp.exp(score(1, 0))
        e1 = jnp.exp(score(1, 1))
        r = 1.0 / (e0 + e1)
        z_ref[1] = (expand(e0 * r) * v[0] +
                    expand(e1 * r) * v[1]).astype(jnp.bfloat16)
        f0 = jnp.exp(score(2, 0))
        f1 = jnp.exp(score(2, 1))
        f2 = jnp.exp(score(2, 2))
        rr = 1.0 / (f0 + f1 + f2)
        z_ref[2] = (expand(f0 * rr) * v[0] + expand(f1 * rr) * v[1] +
                    expand(f2 * rr) * v[2]).astype(jnp.bfloat16)

    return pl.pallas_call(
        body,
        grid=(N // BN,),
        in_specs=[
            pl.BlockSpec((NC, T, BN, D), lambda i: (0, 0, i, 0)),
            pl.BlockSpec((D, 4 * HID), lambda i: (0, 0)),
            pl.BlockSpec((HID, 4 * HID), lambda i: (0, 0)),
            pl.BlockSpec((1, 4 * HID), lambda i: (0, 0)),
            pl.BlockSpec((T, HID), lambda i: (0, 0)),
            pl.BlockSpec((HID, HID), lambda i: (0, 0)),
            pl.BlockSpec((HID, HID), lambda i: (0, 0)),
            pl.BlockSpec((HID, HID), lambda i: (0, 0)),
            pl.BlockSpec((HID, H), lambda i: (0, 0)),
            pl.BlockSpec((H, HID), lambda i: (0, 0)),
        ],
        out_specs=pl.BlockSpec((T, BN, HID), lambda i: (0, i, 0)),
        out_shape=jax.ShapeDtypeStruct((T, N, HID), jnp.bfloat16),
    )(outp, Wi, Wh, b2, pos, Wq, Wk, Wv, HS, EX8)


def _tc_rowsum(ps):
    B1 = 296
    def body(p_ref, s_ref):
        s_ref[...] = jnp.sum(p_ref[...], axis=-1)

    return pl.pallas_call(
        body,
        grid=(NCHL_PAD // B1,),
        in_specs=[pl.BlockSpec((B1, CH, 16), lambda i: (i, 0, 0))],
        out_specs=pl.BlockSpec((B1, CH), lambda i: (i, 0)),
        out_shape=jax.ShapeDtypeStruct((NCHL_PAD, CH), jnp.float32),
    )(ps)


def _blockdiag(a):
    dh = D // H
    idx = jnp.arange(D) // dh
    return jnp.zeros((D, H), jnp.float32).at[jnp.arange(D), idx].set(
        a.reshape(-1))


def kernel(x, edge_index, edge_label_index, W1, a_src1, a_dst1, W2, a_src2,
           a_dst2, Wi, Wh, b_lstm, pos_emb, Wq, Wk, Wv):
    xf = x.reshape(TN, F)
    offs = (jnp.arange(T, dtype=jnp.int32) * N)[:, None]
    gsrc2d = (edge_index[:, 0, :] + offs).reshape(NCHE, CH)
    gdst2d = (edge_index[:, 1, :] + offs).reshape(NCHE, CH)
    lsf = (edge_label_index[:, 0, :] + offs).reshape(TEL)
    ldf = (edge_label_index[:, 1, :] + offs).reshape(TEL)
    ls2d = jnp.pad(lsf, (0, TELP - TEL)).reshape(NCHL_PAD, CH)
    ld2d = jnp.pad(ldf, (0, TELP - TEL)).reshape(NCHL_PAD, CH)

    A1 = jnp.concatenate([_blockdiag(a_src1), _blockdiag(a_dst1)], axis=1)
    A2 = jnp.concatenate([_blockdiag(a_src2), _blockdiag(a_dst2)], axis=1)
    dh = D // H
    colh = jnp.arange(D) // dh
    HS = jnp.zeros((D, H), jnp.float32).at[jnp.arange(D), colh].set(1.0)
    EX8 = HS.T
    zden = jnp.zeros((TN, H), jnp.float32)
    zout = jnp.zeros((N, D), jnp.float32)
    b2 = b_lstm.reshape(1, 4 * HID)
    ldst2d = edge_index[:, 1, :].reshape(NCHE, CH)

    h1, aa1 = _tc_project(xf, W1, A1)
    ex1, den1p = _edge_pass1(aa1, gsrc2d, gdst2d, zden)
    recip1 = _tc_recip_den(den1p)
    coe1p, out1p = _edge_pass2(h1, recip1, ex1, gsrc2d, gdst2d, ldst2d, zout)
    out1p = out1p.reshape(NC, TN, D)

    h2, aa2 = _tc_combine_project(out1p, W2, A2)
    ex2, den2p = _edge_pass1(aa2, gsrc2d, gdst2d, zden)
    recip2 = _tc_recip_den(den2p)
    coe2p, out2p = _edge_pass2(h2, recip2, ex2, gsrc2d, gdst2d, ldst2d, zout)
    out2p = out2p.reshape(NC, TN, D)

    z = _tc_lstm_attn(out2p.reshape(NC, T, N, D), Wi, Wh, b2, pos_emb,
                      Wq, Wk, Wv, HS, EX8)

    ps = _decode(z.reshape(TN, HID), ls2d, ld2d)
    st_flat = _tc_rowsum(ps).reshape(TELP)
    st = st_flat[:TEL].reshape(T, EL)

    coe1 = coe1p.reshape(TE, H).reshape(T, E, H)
    coe2 = coe2p.reshape(TE, H).reshape(T, E, H)
    return (st, coe1, coe2)


# speedup vs baseline: 1.1164x; 1.0090x over previous
import functools
import jax
import jax.numpy as jnp
import numpy as np
from jax import lax
from jax.experimental import pallas as pl
from jax.experimental.pallas import tpu as pltpu
from jax.experimental.pallas import tpu_sc as plsc

T, N, F = 3, 10000, 128
E, EL = 320000, 100000
H = 8
D = 64
HID = 64
TN = T * N

NC, NS = 2, 16
NW = NC * NS
CH = 128
SUP = 5

TE = T * E
NCHE = TE // CH
SUP_L = 2
TEL = T * EL
NCHL_PAD = 2368
TELP = NCHL_PAD * CH
DEC_W = NCHL_PAD // NW

RPT = TN // NS

_SC_PARAMS = pltpu.CompilerParams(
    needs_layout_passes=False, use_tc_tiling_on_sc=False)


def _mesh():
    return plsc.VectorSubcoreMesh(core_axis_name="c", subcore_axis_name="s")


def _edge_pass1(aa, gsrc2d, gdst2d, zden):
    @functools.partial(
        pl.kernel,
        mesh=_mesh(),
        compiler_params=_SC_PARAMS,
        out_type=(jax.ShapeDtypeStruct((NCHE, CH, H), jnp.float32),
                  jax.ShapeDtypeStruct((NC, TN, H), jnp.float32)),
        scratch_types=[
            pltpu.VMEM((SUP, CH), jnp.int32),
            pltpu.VMEM((SUP, CH), jnp.int32),
            pltpu.VMEM((SUP, CH, 2 * H), jnp.float32),
            pltpu.VMEM((SUP, CH, 2 * H), jnp.float32),
            pltpu.VMEM((SUP, CH, H), jnp.float32),
            pltpu.VMEM_SHARED((TN, H), jnp.float32),
            pltpu.SemaphoreType.DMA,
        ],
    )
    def k(aa_hbm, gsrc_hbm, gdst_hbm, zden_hbm, ex_hbm, denp_hbm,
          isrc_v, idst_v, srow_v, drow_v, ex_v, den_sh, sem):
        cid = lax.axis_index("c")
        sid = lax.axis_index("s")
        wid = cid * NS + sid
        pltpu.sync_copy(zden_hbm.at[pl.ds(sid * RPT, RPT)],
                        den_sh.at[pl.ds(sid * RPT, RPT)])
        plsc.subcore_barrier()
        iota = lax.iota(jnp.int32, 16)
        nsup = 46 + jnp.where(wid < 28, 1, 0)
        base_chunk = wid * 230 + jnp.minimum(wid, 28) * SUP

        def step(s, _):
            ch0 = base_chunk + s * SUP
            pltpu.sync_copy(gsrc_hbm.at[pl.ds(ch0, SUP)], isrc_v)
            pltpu.sync_copy(gdst_hbm.at[pl.ds(ch0, SUP)], idst_v)
            descs = []
            for j in range(SUP):
                descs.append(pltpu.async_copy(
                    aa_hbm.at[isrc_v.at[j]], srow_v.at[j], sem))
                descs.append(pltpu.async_copy(
                    aa_hbm.at[idst_v.at[j]], drow_v.at[j], sem))
            for dsc in descs:
                dsc.wait()
            for j in range(SUP):
                jv = jnp.full((16,), j, jnp.int32)

                def body(i, _):
                    kk = i * 16 + iota
                    cv = kk >> 3
                    hv = kk & 7
                    sv = plsc.load_gather(srow_v, [jv, cv, hv])
                    dv = plsc.load_gather(drow_v, [jv, cv, hv + 8])
                    sm = sv + dv
                    val = jnp.exp(jnp.maximum(sm, 0.2 * sm))
                    plsc.store_scatter(ex_v, [jv, cv, hv], val)
                    return 0
                lax.fori_loop(0, CH * H // 16, body, 0, unroll=4)
            pltpu.sync_copy(ex_v, ex_hbm.at[pl.ds(ch0, SUP)])
            for j in range(SUP):
                pltpu.sync_copy(ex_v.at[j], den_sh.at[idst_v.at[j]], add=True)
            return 0
        lax.fori_loop(0, nsup, step, 0)
        plsc.subcore_barrier()
        pltpu.sync_copy(den_sh.at[pl.ds(sid * RPT, RPT)],
                        denp_hbm.at[cid, pl.ds(sid * RPT, RPT)])

    return k(aa, gsrc2d, gdst2d, zden)


NCHT = E // CH
RPN = N // NS


def _edge_pass2(h_table, recip, exbuf, gsrc2d, gdst2d, ldst2d, zout):
    @functools.partial(
        pl.kernel,
        mesh=_mesh(),
        compiler_params=_SC_PARAMS,
        out_type=(jax.ShapeDtypeStruct((NCHE, CH, H), jnp.float32),
                  jax.ShapeDtypeStruct((NC, T, N, D), jnp.float32)),
        scratch_types=[
            pltpu.VMEM((SUP, CH), jnp.int32),
            pltpu.VMEM((SUP, CH), jnp.int32),
            pltpu.VMEM((SUP, CH), jnp.int32),
            pltpu.VMEM((SUP, CH, D), jnp.bfloat16),
            pltpu.VMEM((SUP, CH, D), jnp.float32),
            pltpu.VMEM((SUP, CH, H), jnp.float32),
            pltpu.VMEM((SUP, CH, H), jnp.float32),
            pltpu.VMEM_SHARED((N, D), jnp.float32),
            pltpu.SemaphoreType.DMA,
        ],
    )
    def k(h_hbm, recip_hbm, ex_hbm, gsrc_hbm, gdst_hbm, ldst_hbm, zout_hbm,
          coe_hbm, outp_hbm,
          isrc_v, idst_v, ildst_v, hrow_v, arow_v, rrow_v, ex_v, out_sh, sem):
        cid = lax.axis_index("c")
        sid = lax.axis_index("s")
        wid = cid * NS + sid
        iota = lax.iota(jnp.int32, 16)
        nsup = 15 + jnp.where(wid < 20, 1, 0)
        base_local = wid * 75 + jnp.minimum(wid, 20) * SUP

        def tbody(t, _):
            pltpu.sync_copy(zout_hbm.at[pl.ds(sid * RPN, RPN)],
                            out_sh.at[pl.ds(sid * RPN, RPN)])
            plsc.subcore_barrier()

            def step(s, _):
                ch0 = t * NCHT + base_local + s * SUP
                pltpu.sync_copy(gsrc_hbm.at[pl.ds(ch0, SUP)], isrc_v)
                pltpu.sync_copy(gdst_hbm.at[pl.ds(ch0, SUP)], idst_v)
                pltpu.sync_copy(ldst_hbm.at[pl.ds(ch0, SUP)], ildst_v)
                pltpu.sync_copy(ex_hbm.at[pl.ds(ch0, SUP)], ex_v)
                descs = []
                for j in range(SUP):
                    descs.append(pltpu.async_copy(
                        h_hbm.at[isrc_v.at[j]], hrow_v.at[j], sem))
                    descs.append(pltpu.async_copy(
                        recip_hbm.at[idst_v.at[j]], rrow_v.at[j], sem))
                for dsc in descs:
                    dsc.wait()
                for j in range(SUP):
                    jv = jnp.full((16,), j, jnp.int32)

                    def body(i, _):
                        kk = i * 16 + iota
                        cv = kk >> 3
                        hv = kk & 7
                        ev = plsc.load_gather(ex_v, [jv, cv, hv])
                        rv = plsc.load_gather(rrow_v, [jv, cv, hv])
                        plsc.store_scatter(ex_v, [jv, cv, hv], ev * rv)
                        return 0
                    lax.fori_loop(0, CH * H // 16, body, 0, unroll=4)
                pltpu.sync_copy(ex_v, coe_hbm.at[pl.ds(ch0, SUP)])
                for j in range(SUP):
                    jv = jnp.full((16,), j, jnp.int32)

                    def body2(c, _):
                        cv = jnp.full((16,), c, jnp.int32)
                        for kq in range(2):
                            hb = hrow_v[j, c, pl.ds(kq * 32, 32)]
                            lo, hi = plsc.unpack(
                                hb, format=plsc.PackFormat.INTERLEAVED)
                            av = plsc.load_gather(
                                ex_v, [jv, cv, 4 * kq + (iota >> 2)])
                            de = kq * 32 + 2 * iota
                            plsc.store_scatter(arow_v, [jv, cv, de], lo * av)
                            plsc.store_scatter(arow_v, [jv, cv, de + 1],
                                               hi * av)
                        return 0
                    lax.fori_loop(0, CH, body2, 0, unroll=4)
                    pltpu.sync_copy(arow_v.at[j], out_sh.at[ildst_v.at[j]],
                                    add=True)
                return 0
            lax.fori_loop(0, nsup, step, 0)
            plsc.subcore_barrier()
            pltpu.sync_copy(out_sh.at[pl.ds(sid * RPN, RPN)],
                            outp_hbm.at[cid, t, pl.ds(sid * RPN, RPN)])
            return 0
        lax.fori_loop(0, T, tbody, 0)

    return k(h_table, recip, exbuf, gsrc2d, gdst2d, ldst2d, zout)


def _decode(z2d, ls2d, ld2d):
    @functools.partial(
        pl.kernel,
        mesh=_mesh(),
        compiler_params=_SC_PARAMS,
        out_type=jax.ShapeDtypeStruct((NCHL_PAD, CH, 16), jnp.float32),
        scratch_types=[
            pltpu.VMEM((SUP_L, CH), jnp.int32),
            pltpu.VMEM((SUP_L, CH), jnp.int32),
            pltpu.VMEM((SUP_L, CH, D), jnp.bfloat16),
            pltpu.VMEM((SUP_L, CH, D), jnp.bfloat16),
            pltpu.VMEM((SUP_L, CH, 16), jnp.float32),
            pltpu.SemaphoreType.DMA,
        ],
    )
    def k(z_hbm, ls_hbm, ld_hbm, ps_hbm, il_v, id_v, zs_v, zd_v, ps_v, sem):
        cid = lax.axis_index("c")
        sid = lax.axis_index("s")
        wid = cid * NS + sid
        base_chunk = wid * DEC_W

        def step(s, _):
            ch0 = base_chunk + s * SUP_L
            pltpu.sync_copy(ls_hbm.at[pl.ds(ch0, SUP_L)], il_v)
            pltpu.sync_copy(ld_hbm.at[pl.ds(ch0, SUP_L)], id_v)
            descs = []
            for j in range(SUP_L):
                descs.append(pltpu.async_copy(
                    z_hbm.at[il_v.at[j]], zs_v.at[j], sem))
                descs.append(pltpu.async_copy(
                    z_hbm.at[id_v.at[j]], zd_v.at[j], sem))
            for dsc in descs:
                dsc.wait()
            for j in range(SUP_L):
                def body(c, _):
                    acc = None
                    for kq in range(2):
                        sb = zs_v[j, c, pl.ds(kq * 32, 32)]
                        db = zd_v[j, c, pl.ds(kq * 32, 32)]
                        slo, shi = plsc.unpack(
                            sb, format=plsc.PackFormat.INTERLEAVED)
                        dlo, dhi = plsc.unpack(
                            db, format=plsc.PackFormat.INTERLEAVED)
                        p = slo * dlo + shi * dhi
                        acc = p if acc is None else acc + p
                    ps_v[j, c, :] = acc
                    return 0
                lax.fori_loop(0, CH, body, 0, unroll=2)
            pltpu.sync_copy(ps_v, ps_hbm.at[pl.ds(ch0, SUP_L)])
            return 0
        lax.fori_loop(0, DEC_W // SUP_L, step, 0)

    return k(z2d, ls2d, ld2d)


BN = 1000


def _tc_project(xf, W, A):
    Fin = xf.shape[1]

    def body(x_ref, w_ref, a_ref, h_ref, aa_ref):
        h = jnp.dot(x_ref[...], w_ref[...], preferred_element_type=jnp.float32)
        h_ref[...] = h.astype(jnp.bfloat16)
        aa_ref[...] = jnp.dot(h, a_ref[...], preferred_element_type=jnp.float32)

    return pl.pallas_call(
        body,
        grid=(TN // BN,),
        in_specs=[
            pl.BlockSpec((BN, Fin), lambda i: (i, 0)),
            pl.BlockSpec((Fin, D), lambda i: (0, 0)),
            pl.BlockSpec((D, 2 * H), lambda i: (0, 0)),
        ],
        out_specs=[
            pl.BlockSpec((BN, D), lambda i: (i, 0)),
            pl.BlockSpec((BN, 2 * H), lambda i: (i, 0)),
        ],
        out_shape=[
            jax.ShapeDtypeStruct((TN, D), jnp.bfloat16),
            jax.ShapeDtypeStruct((TN, 2 * H), jnp.float32),
        ],
    )(xf, W, A)


def _tc_combine_project(outp, W, A):
    def body(p_ref, w_ref, a_ref, h_ref, aa_ref):
        s = p_ref[0] + p_ref[1]
        o = jnp.where(s > 0, s, jnp.exp(jnp.minimum(s, 0.0)) - 1.0)
        h = jnp.dot(o, w_ref[...], preferred_element_type=jnp.float32)
        h_ref[...] = h.astype(jnp.bfloat16)
        aa_ref[...] = jnp.dot(h, a_ref[...], preferred_element_type=jnp.float32)

    return pl.pallas_call(
        body,
        grid=(TN // BN,),
        in_specs=[
            pl.BlockSpec((NC, BN, D), lambda i: (0, i, 0)),
            pl.BlockSpec((D, D), lambda i: (0, 0)),
            pl.BlockSpec((D, 2 * H), lambda i: (0, 0)),
        ],
        out_specs=[
            pl.BlockSpec((BN, D), lambda i: (i, 0)),
            pl.BlockSpec((BN, 2 * H), lambda i: (i, 0)),
        ],
        out_shape=[
            jax.ShapeDtypeStruct((TN, D), jnp.bfloat16),
            jax.ShapeDtypeStruct((TN, 2 * H), jnp.float32),
        ],
    )(outp, W, A)


def _tc_recip_den(denp):
    def body(d_ref, r_ref):
        r_ref[...] = 1.0 / (d_ref[0] + d_ref[1] + 1e-16)

    BR = 3000
    return pl.pallas_call(
        body,
        grid=(TN // BR,),
        in_specs=[pl.BlockSpec((NC, BR, H), lambda i: (0, i, 0))],
        out_specs=pl.BlockSpec((BR, H), lambda i: (i, 0)),
        out_shape=jax.ShapeDtypeStruct((TN, H), jnp.float32),
    )(denp)


def _tc_lstm_attn(outp, Wi, Wh, b2, pos, Wq, Wk, Wv, HS, EX8):
    inv_sqrt_dh = 1.0 / np.sqrt(H)

    def body(p_ref, wi_ref, wh_ref, b_ref, pos_ref, wq_ref, wk_ref, wv_ref,
             hs_ref, ex_ref, z_ref):
        wi = wi_ref[...]
        wh = wh_ref[...]
        b = b_ref[...]
        hst = jnp.zeros((BN, HID), jnp.float32)
        cst = jnp.zeros((BN, HID), jnp.float32)
        seq = []
        for t in range(T):
            o2 = jnp.maximum(p_ref[0, t] + p_ref[1, t], 0.0)
            g = (jnp.dot(o2, wi, preferred_element_type=jnp.float32) +
                 jnp.dot(hst, wh, preferred_element_type=jnp.float32) + b)
            ig = jax.nn.sigmoid(g[:, 0 * HID:1 * HID])
            fg = jax.nn.sigmoid(g[:, 1 * HID:2 * HID])
            gg = jnp.tanh(g[:, 2 * HID:3 * HID])
            og = jax.nn.sigmoid(g[:, 3 * HID:4 * HID])
            cst = fg * cst + ig * gg
            hst = og * jnp.tanh(cst)
            seq.append(hst)
        hs = hs_ref[...]
        ex8 = ex_ref[...]
        xx = [seq[t] + pos_ref[t, :][None, :] for t in range(T)]
        q = [jnp.dot(xx[t], wq_ref[...], preferred_element_type=jnp.float32)
             for t in range(T)]
        kk = [jnp.dot(xx[t], wk_ref[...], preferred_element_type=jnp.float32)
              for t in range(T)]
        v = [jnp.dot(xx[t], wv_ref[...], preferred_element_type=jnp.float32)
             for t in range(T)]

        def score(qt, kt):
            return jnp.dot(q[qt] * kk[kt], hs,
                           preferred_element_type=jnp.float32) * inv_sqrt_dh

        def expand(a):
            return jnp.dot(a, ex8, preferred_element_type=jnp.float32)

        z_ref[0] = v[0].astype(jnp.bfloat16)
        e0 = jnp.exp(score(1, 0))
        e1 = jnp.exp(score(1, 1))
        r = 1.0 / (e0 + e1)
        z_ref[1] = (expand(e0 * r) * v[0] +
                    expand(e1 * r) * v[1]).astype(jnp.bfloat16)
        f0 = jnp.exp(score(2, 0))
        f1 = jnp.exp(score(2, 1))
        f2 = jnp.exp(score(2, 2))
        rr = 1.0 / (f0 + f1 + f2)
        z_ref[2] = (expand(f0 * rr) * v[0] + expand(f1 * rr) * v[1] +
                    expand(f2 * rr) * v[2]).astype(jnp.bfloat16)

    return pl.pallas_call(
        body,
        grid=(N // BN,),
        in_specs=[
            pl.BlockSpec((NC, T, BN, D), lambda i: (0, 0, i, 0)),
            pl.BlockSpec((D, 4 * HID), lambda i: (0, 0)),
            pl.BlockSpec((HID, 4 * HID), lambda i: (0, 0)),
            pl.BlockSpec((1, 4 * HID), lambda i: (0, 0)),
            pl.BlockSpec((T, HID), lambda i: (0, 0)),
            pl.BlockSpec((HID, HID), lambda i: (0, 0)),
            pl.BlockSpec((HID, HID), lambda i: (0, 0)),
            pl.BlockSpec((HID, HID), lambda i: (0, 0)),
            pl.BlockSpec((HID, H), lambda i: (0, 0)),
            pl.BlockSpec((H, HID), lambda i: (0, 0)),
        ],
        out_specs=pl.BlockSpec((T, BN, HID), lambda i: (0, i, 0)),
        out_shape=jax.ShapeDtypeStruct((T, N, HID), jnp.bfloat16),
    )(outp, Wi, Wh, b2, pos, Wq, Wk, Wv, HS, EX8)


def _tc_rowsum(ps):
    B1 = 296
    def body(p_ref, s_ref):
        s_ref[...] = jnp.sum(p_ref[...], axis=-1)

    return pl.pallas_call(
        body,
        grid=(NCHL_PAD // B1,),
        in_specs=[pl.BlockSpec((B1, CH, 16), lambda i: (i, 0, 0))],
        out_specs=pl.BlockSpec((B1, CH), lambda i: (i, 0)),
        out_shape=jax.ShapeDtypeStruct((NCHL_PAD, CH), jnp.float32),
    )(ps)


def _blockdiag(a):
    dh = D // H
    idx = jnp.arange(D) // dh
    return jnp.zeros((D, H), jnp.float32).at[jnp.arange(D), idx].set(
        a.reshape(-1))


def kernel(x, edge_index, edge_label_index, W1, a_src1, a_dst1, W2, a_src2,
           a_dst2, Wi, Wh, b_lstm, pos_emb, Wq, Wk, Wv):
    xf = x.reshape(TN, F)
    offs = (jnp.arange(T, dtype=jnp.int32) * N)[:, None]
    gsrc2d = (edge_index[:, 0, :] + offs).reshape(NCHE, CH)
    gdst2d = (edge_index[:, 1, :] + offs).reshape(NCHE, CH)
    lsf = (edge_label_index[:, 0, :] + offs).reshape(TEL)
    ldf = (edge_label_index[:, 1, :] + offs).reshape(TEL)
    ls2d = jnp.pad(lsf, (0, TELP - TEL)).reshape(NCHL_PAD, CH)
    ld2d = jnp.pad(ldf, (0, TELP - TEL)).reshape(NCHL_PAD, CH)

    A1 = jnp.concatenate([_blockdiag(a_src1), _blockdiag(a_dst1)], axis=1)
    A2 = jnp.concatenate([_blockdiag(a_src2), _blockdiag(a_dst2)], axis=1)
    dh = D // H
    colh = jnp.arange(D) // dh
    HS = jnp.zeros((D, H), jnp.float32).at[jnp.arange(D), colh].set(1.0)
    EX8 = HS.T
    zden = jnp.zeros((TN, H), jnp.float32)
    zout = jnp.zeros((N, D), jnp.float32)
    b2 = b_lstm.reshape(1, 4 * HID)
    ldst2d = edge_index[:, 1, :].reshape(NCHE, CH)

    h1, aa1 = _tc_project(xf, W1, A1)
    ex1, den1p = _edge_pass1(aa1, gsrc2d, gdst2d, zden)
    recip1 = _tc_recip_den(den1p)
    coe1p, out1p = _edge_pass2(h1, recip1, ex1, gsrc2d, gdst2d, ldst2d, zout)
    out1p = out1p.reshape(NC, TN, D)

    h2, aa2 = _tc_combine_project(out1p, W2, A2)
    ex2, den2p = _edge_pass1(aa2, gsrc2d, gdst2d, zden)
    recip2 = _tc_recip_den(den2p)
    coe2p, out2p = _edge_pass2(h2, recip2, ex2, gsrc2d, gdst2d, ldst2d, zout)
    out2p = out2p.reshape(NC, TN, D)

    z = _tc_lstm_attn(out2p.reshape(NC, T, N, D), Wi, Wh, b2, pos_emb,
                      Wq, Wk, Wv, HS, EX8)

    ps = _decode(z.reshape(TN, HID), ls2d, ld2d)
    st_flat = _tc_rowsum(ps).reshape(TELP)
    st = st_flat[:TEL].reshape(T, EL)

    coe1 = coe1p.reshape(TE, H).reshape(T, E, H)
    coe2 = coe2p.reshape(TE, H).reshape(T, E, H)
    return (st, coe1, coe2)
